# Initial kernel scaffold; baseline (speedup 1.0000x reference)
import jax
import jax.numpy as jnp
from jax.experimental import pallas as pl


def kernel(x, masked_nodes, pos_edge_index, neg_edge_index, edge_index, W1, b1, W2, b2, s1w, s1b, s2w, s2b, s3w, s3b):
    raise NotImplementedError("write your pallas kernel here")


# speedup vs baseline: 19.1793x; 19.1793x over previous
import functools

import jax
import jax.numpy as jnp
from jax import lax
from jax.experimental import pallas as pl
from jax.experimental.pallas import tpu as pltpu
from jax.experimental.pallas import tpu_sc as plsc

N = 10000
D_IN = 128
HID = 128
NCLS = 64
E = 320000
TE = 320000

NC = 2
NS = 16
NW = NC * NS
CHUNK = 128
ECH = 79
E_PAD = NW * ECH * CHUNK
N_PAD = 10112
ROWS_PER_TILE = N_PAD // NS
E_PER_TILE = E_PAD // NW
TE_PAD = E_PAD
TE_PER_TILE = TE_PAD // NW

_MESH = plsc.VectorSubcoreMesh(
    core_axis_name="c", subcore_axis_name="s", num_cores=NC, num_subcores=NS)

_SC_PARAMS = pltpu.CompilerParams(needs_layout_passes=False,
                                  use_tc_tiling_on_sc=False)


@functools.partial(
    pl.kernel,
    out_type=jax.ShapeDtypeStruct((NW * N_PAD,), jnp.float32),
    mesh=_MESH,
    compiler_params=_SC_PARAMS,
    scratch_types=[
        pltpu.VMEM((E_PER_TILE,), jnp.int32),
        pltpu.VMEM((N_PAD,), jnp.float32),
    ],
)
def _deg_kernel(dst_hbm, out_hbm, idx_v, cnt_v):
    c = lax.axis_index("c")
    s = lax.axis_index("s")
    w = c * NS + s
    pltpu.sync_copy(dst_hbm.at[pl.ds(w * E_PER_TILE, E_PER_TILE)], idx_v)
    zero16 = jnp.zeros((16,), jnp.float32)

    def zbody(i, _):
        cnt_v[pl.ds(i * 16, 16)] = zero16
        return 0

    lax.fori_loop(0, N_PAD // 16, zbody, 0)
    one16 = jnp.ones((16,), jnp.float32)

    def body(i, _):
        idx = idx_v[pl.ds(i * 16, 16)]
        plsc.addupdate_scatter(cnt_v, [idx], one16)
        return 0

    lax.fori_loop(0, E_PER_TILE // 16, body, 0)
    pltpu.sync_copy(cnt_v, out_hbm.at[pl.ds(w * N_PAD, N_PAD)])


def _make_gather_scatter(D):
    @functools.partial(
        pl.kernel,
        out_type=jax.ShapeDtypeStruct((NC, N_PAD, D), jnp.float32),
        mesh=_MESH,
        compiler_params=_SC_PARAMS,
        scratch_types=[
            pltpu.VMEM((ECH, CHUNK), jnp.int32),
            pltpu.VMEM((ECH, CHUNK), jnp.int32),
            pltpu.VMEM((CHUNK, D), jnp.float32),
            pltpu.VMEM_SHARED((N_PAD, D), jnp.float32),
            pltpu.SemaphoreType.DMA,
        ],
    )
    def k(src_hbm, dst_hbm, h_hbm, zeros_hbm, out_hbm,
          src_v, dst_v, rows_v, acc, sem):
        c = lax.axis_index("c")
        s = lax.axis_index("s")
        pltpu.sync_copy(src_hbm.at[c, s], src_v)
        pltpu.sync_copy(dst_hbm.at[c, s], dst_v)
        row0 = s * ROWS_PER_TILE
        pltpu.sync_copy(zeros_hbm.at[pl.ds(row0, ROWS_PER_TILE)],
                        acc.at[pl.ds(row0, ROWS_PER_TILE)])
        plsc.subcore_barrier()

        def body(j, _):
            pltpu.async_copy(h_hbm.at[src_v.at[j]], rows_v, sem).wait()
            pltpu.sync_copy(rows_v, acc.at[dst_v.at[j]], add=True)
            return 0

        lax.fori_loop(0, ECH, body, 0)
        plsc.subcore_barrier()
        pltpu.sync_copy(acc.at[pl.ds(row0, ROWS_PER_TILE)],
                        out_hbm.at[c, pl.ds(row0, ROWS_PER_TILE)])

    return k


_gs_hid = _make_gather_scatter(HID)
_gs_cls = _make_gather_scatter(NCLS)


@functools.partial(
    pl.kernel,
    out_type=(jax.ShapeDtypeStruct((TE_PAD,), jnp.float32),
              jax.ShapeDtypeStruct((NW * 128,), jnp.float32)),
    mesh=_MESH,
    compiler_params=_SC_PARAMS,
    scratch_types=[
        pltpu.VMEM((N,), jnp.float32),
        pltpu.VMEM((N,), jnp.float32),
        pltpu.VMEM((N,), jnp.float32),
        pltpu.VMEM((TE_PER_TILE,), jnp.int32),
        pltpu.VMEM((TE_PER_TILE,), jnp.int32),
        pltpu.VMEM((TE_PER_TILE,), jnp.float32),
        pltpu.VMEM((128,), jnp.float32),
    ],
)
def _score_kernel(qa_hbm, qb_hbm, r_hbm, t0_hbm, t1_hbm, o3_hbm, sums_hbm,
                  qa_v, qb_v, r_v, t0_v, t1_v, o3_v, acc_v):
    c = lax.axis_index("c")
    s = lax.axis_index("s")
    w = c * NS + s
    base = w * TE_PER_TILE
    pltpu.sync_copy(qa_hbm, qa_v)
    pltpu.sync_copy(qb_hbm, qb_v)
    pltpu.sync_copy(r_hbm, r_v)
    pltpu.sync_copy(t0_hbm.at[pl.ds(base, TE_PER_TILE)], t0_v)
    pltpu.sync_copy(t1_hbm.at[pl.ds(base, TE_PER_TILE)], t1_v)

    def body(i, acc):
        i0 = t0_v[pl.ds(i * 16, 16)]
        i1 = t1_v[pl.ds(i * 16, 16)]
        g_qa = plsc.load_gather(qa_v, [i0])
        g_qb = plsc.load_gather(qb_v, [i1])
        o3_v[pl.ds(i * 16, 16)] = jnp.maximum(g_qa - g_qb, 0.0)
        g_r0 = plsc.load_gather(r_v, [i0])
        g_r1 = plsc.load_gather(r_v, [i1])
        return acc + (g_r0 - g_r1)

    acc = lax.fori_loop(0, TE_PER_TILE // 16, body,
                        jnp.zeros((16,), jnp.float32))
    zero16 = jnp.zeros((16,), jnp.float32)

    def zsum(i, _):
        acc_v[pl.ds(i * 16, 16)] = zero16
        return 0

    lax.fori_loop(0, 8, zsum, 0)
    acc_v[pl.ds(0, 16)] = acc
    pltpu.sync_copy(o3_v, o3_hbm.at[pl.ds(base, TE_PER_TILE)])
    pltpu.sync_copy(acc_v, sums_hbm.at[pl.ds(w * 128, 128)])


def _tc1_body(x_ref, w1_ref, s1w_ref, cnt_ref, hp_ref, p1_ref, dinv_ref):
    cs = jnp.sum(cnt_ref[...], axis=0)
    deg = cs[:N] + 1.0
    y = lax.rsqrt(deg)
    dinv = y * (1.5 - 0.5 * deg * y * y)
    dinv_ref[...] = dinv
    xx = x_ref[...]
    h = jnp.dot(xx, w1_ref[...], precision=lax.Precision.HIGHEST,
                preferred_element_type=jnp.float32)
    hp_ref[...] = h * dinv[:, None]
    p1_ref[...] = jnp.sum(xx * s1w_ref[...][None, :], axis=1)


def _tc3_body(p_ref, hp_ref, dinv_ref, b1_ref, w2_ref, s2w_ref,
              h2p_ref, p2_ref):
    agg = p_ref[0, :N, :] + p_ref[1, :N, :] + hp_ref[...]
    dinv = dinv_ref[...]
    h1 = jnp.maximum(agg * dinv[:, None] + b1_ref[...][None, :], 0.0)
    h2r = jnp.dot(h1, w2_ref[...], precision=lax.Precision.HIGHEST,
                  preferred_element_type=jnp.float32)
    h2p_ref[...] = h2r * dinv[:, None]
    p2_ref[...] = jnp.sum(h1 * s2w_ref[...][None, :], axis=1)


def _tc4_body(p_ref, h2p_ref, dinv_ref, b2_ref, s3w_ref, s3b_ref,
              p1_ref, p2_ref, qa_ref, qb_ref, r_ref):
    agg = p_ref[0, :N, :] + p_ref[1, :N, :] + h2p_ref[...]
    h2 = agg * dinv_ref[...][:, None] + b2_ref[...][None, :]
    q = jnp.sum(h2 * s3w_ref[...][None, :], axis=1)
    qa_ref[...] = q + s3b_ref[...]
    qb_ref[...] = q
    r_ref[...] = p1_ref[...] + p2_ref[...] + q


_tc1 = pl.pallas_call(
    _tc1_body,
    out_shape=(jax.ShapeDtypeStruct((N, HID), jnp.float32),
               jax.ShapeDtypeStruct((N,), jnp.float32),
               jax.ShapeDtypeStruct((N,), jnp.float32)),
)

_tc3 = pl.pallas_call(
    _tc3_body,
    out_shape=(jax.ShapeDtypeStruct((N, NCLS), jnp.float32),
               jax.ShapeDtypeStruct((N,), jnp.float32)),
)

_tc4 = pl.pallas_call(
    _tc4_body,
    out_shape=(jax.ShapeDtypeStruct((N,), jnp.float32),
               jax.ShapeDtypeStruct((N,), jnp.float32),
               jax.ShapeDtypeStruct((N,), jnp.float32)),
)


def kernel(x, masked_nodes, pos_edge_index, neg_edge_index, edge_index,
           W1, b1, W2, b2, s1w, s1b, s2w, s2b, s3w, s3b):
    src = edge_index[0].astype(jnp.int32)
    dst = edge_index[1].astype(jnp.int32)
    pad_src = jnp.zeros((E_PAD - E,), jnp.int32)
    pad_dst = jnp.full((E_PAD - E,), N, jnp.int32)
    src_p = jnp.concatenate([src, pad_src]).reshape(NC, NS, ECH, CHUNK)
    dst_flat = jnp.concatenate([dst, pad_dst])
    dst_p = dst_flat.reshape(NC, NS, ECH, CHUNK)
    pad_t = jnp.zeros((TE_PAD - TE,), jnp.int32)
    t0 = jnp.concatenate(
        [pos_edge_index[0], neg_edge_index[0], pad_t]).astype(jnp.int32)
    t1 = jnp.concatenate(
        [pos_edge_index[1], neg_edge_index[1], pad_t]).astype(jnp.int32)

    cnt = _deg_kernel(dst_flat).reshape(NW, N_PAD)
    hp, p1, dinv = _tc1(x, W1, s1w, cnt)
    parts1 = _gs_hid(src_p, dst_p, hp, jnp.zeros((N_PAD, HID), jnp.float32))
    h2p, p2 = _tc3(parts1, hp, dinv, b1, W2, s2w)
    parts2 = _gs_cls(src_p, dst_p, h2p, jnp.zeros((N_PAD, NCLS), jnp.float32))
    qa, qb, r = _tc4(parts2, h2p, dinv, b2, s3w, s3b, p1, p2)
    o3_pad, sums = _score_kernel(qa, qb, r, t0, t1)
    score_loss = jnp.sum(sums) / TE
    return (o3_pad[:TE], score_loss)
